# baseline (speedup 1.0000x reference)
import functools

import jax
import jax.numpy as jnp
from jax import lax
from jax.experimental import pallas as pl
from jax.experimental.pallas import tpu as pltpu
from jax.experimental.pallas import tpu_sc as plsc

NUM_P, NUM_S, NUM_E, NUM_L = 50000, 10000, 800000, 200000
D_EMB, D_H = 128, 64

NC, NSUB, L = 2, 16, 16
NW = NC * NSUB

EG = NUM_E // 128
EGP = 6272
DEG_P_PAD = 50176
DEG_S_PAD = 10240
P_ACC = NUM_P + 16
S_ACC = NUM_S + 16
NL_PAD = 200704
LG = NL_PAD // 128

_mesh = lambda: plsc.VectorSubcoreMesh(
    core_axis_name="c", subcore_axis_name="s", num_cores=NC, num_subcores=NSUB)
_f32 = jnp.float32
_sc_params = lambda: pltpu.CompilerParams(
    needs_layout_passes=False, use_tc_tiling_on_sc=False)


def _deg_kernel(src2, dst2, zeros_deg):
    n_my = EGP // NW

    @functools.partial(
        pl.kernel,
        out_type=[
            jax.ShapeDtypeStruct((NW, DEG_P_PAD), _f32),
            jax.ShapeDtypeStruct((NW, DEG_S_PAD), _f32),
        ],
        mesh=_mesh(),
        compiler_params=_sc_params(),
        scratch_types=[
            pltpu.VMEM((DEG_P_PAD,), _f32),
            pltpu.VMEM((DEG_S_PAD,), _f32),
            pltpu.VMEM((2, 128), jnp.int32),
            pltpu.VMEM((2, 128), jnp.int32),
            pltpu.SemaphoreType.DMA,
            pltpu.SemaphoreType.DMA,
            pltpu.SemaphoreType.DMA,
            pltpu.SemaphoreType.DMA,
        ],
    )
    def k(src_h, dst_h, z_h, hp_out, hs_out, hist_p, hist_s, sbuf, dbuf,
          ss0, ss1, ds0, ds1):
        cid = lax.axis_index("c")
        sid = lax.axis_index("s")
        w = cid * NSUB + sid
        ssem = (ss0, ss1)
        dsem = (ds0, ds1)
        pltpu.sync_copy(z_h.at[pl.ds(0, DEG_P_PAD)], hist_p)
        pltpu.sync_copy(z_h.at[pl.ds(0, DEG_S_PAD)], hist_s)
        ones = jnp.ones((L,), _f32)

        def issue(i, j):
            g = jnp.minimum(w + i * NW, EGP - 1)
            pltpu.async_copy(src_h.at[g], sbuf.at[j], ssem[j])
            pltpu.async_copy(dst_h.at[g], dbuf.at[j], dsem[j])

        def wait(j):
            pltpu.make_async_copy(src_h.at[0], sbuf.at[j], ssem[j]).wait()
            pltpu.make_async_copy(dst_h.at[0], dbuf.at[j], dsem[j]).wait()

        issue(0, 0)
        issue(1, 1)

        def body(k2, _):
            for j in range(2):
                i = 2 * k2 + j
                wait(j)
                for v in range(128 // L):
                    plsc.addupdate_scatter(
                        hist_p, [sbuf[j, pl.ds(v * L, L)]], ones)
                    plsc.addupdate_scatter(
                        hist_s, [dbuf[j, pl.ds(v * L, L)]], ones)
                issue(i + 2, j)
            return 0

        lax.fori_loop(0, n_my // 2, body, 0)
        wait(0)
        wait(1)
        pltpu.sync_copy(hist_p, hp_out.at[w])
        pltpu.sync_copy(hist_s, hs_out.at[w])

    return k(src2, dst2, zeros_deg)


def _recip_kernel(parts_p, parts_s):
    cpt_p = DEG_P_PAD // NW
    cpt_s = DEG_S_PAD // NW

    @functools.partial(
        pl.kernel,
        out_type=[
            jax.ShapeDtypeStruct((DEG_P_PAD,), _f32),
            jax.ShapeDtypeStruct((DEG_S_PAD,), _f32),
        ],
        mesh=_mesh(),
        compiler_params=_sc_params(),
        scratch_types=[
            pltpu.VMEM((NW, cpt_p), _f32),
            pltpu.VMEM((cpt_p,), _f32),
            pltpu.VMEM((NW, cpt_s), _f32),
            pltpu.VMEM((cpt_s,), _f32),
        ],
    )
    def k(pp_h, ps_h, op_h, os_h, bufp, rbufp, bufs, rbufs):
        cid = lax.axis_index("c")
        sid = lax.axis_index("s")
        w = cid * NSUB + sid

        for parts_h, out_h, buf, rbuf, cpt in (
                (pp_h, op_h, bufp, rbufp, cpt_p),
                (ps_h, os_h, bufs, rbufs, cpt_s)):
            for j in range(NW):
                pltpu.sync_copy(parts_h.at[j, pl.ds(w * cpt, cpt)], buf.at[j])

            def body(c, _, buf=buf, rbuf=rbuf):
                off = c * L
                s = buf[0, pl.ds(off, L)]
                for j in range(1, NW):
                    s = s + buf[j, pl.ds(off, L)]
                rbuf[pl.ds(off, L)] = 1.0 / jnp.maximum(s, 1.0)
                return 0

            lax.fori_loop(0, cpt // L, body, 0)
            pltpu.sync_copy(rbuf, out_h.at[pl.ds(w * cpt, cpt)])

    return k(parts_p, parts_s)


def _segsum_layer(tbl_p, tbl_s, es_g, ed_g, es_s, ed_s, zeros_acc):
    n_my = EGP // NW
    RB = 4
    dt = jnp.bfloat16

    @functools.partial(
        pl.kernel,
        out_type=jax.ShapeDtypeStruct((NC, NUM_S + NUM_P, D_H), dt),
        mesh=_mesh(),
        compiler_params=_sc_params(),
        scratch_types=[
            pltpu.VMEM_SHARED((P_ACC, D_H), dt),
            pltpu.VMEM((4, 128), jnp.int32),
            pltpu.VMEM((4, 128), jnp.int32),
            pltpu.VMEM((RB, 128, D_H), dt),
        ] + [pltpu.SemaphoreType.DMA] * (8 + 2 * RB),
    )
    def k(tp_h, ts_h, esg_h, edg_h, ess_h, eds_h, z_h, out_h,
          acc, gbufs, sbufs, rows, *sems):
        cid = lax.axis_index("c")
        sid = lax.axis_index("s")
        sem_ig, sem_is = sems[0:4], sems[4:8]
        sem_g, sem_sc = sems[8:8 + RB], sems[8 + RB:8 + 2 * RB]
        w = cid * NSUB + sid

        def zero(alloc_rows):
            rpt = alloc_rows // NSUB
            pltpu.sync_copy(z_h.at[pl.ds(sid * rpt, rpt)],
                            acc.at[pl.ds(sid * rpt, rpt)])

        def run_pass(tbl_h, g_h, s_h):
            def gid(i):
                return jnp.minimum(w + i * NW, EGP - 1)

            def idx_issue(j, i):
                g = gid(i)
                pltpu.async_copy(g_h.at[g], gbufs.at[j], sem_ig[j])
                pltpu.async_copy(s_h.at[g], sbufs.at[j], sem_is[j])

            def idx_wait(j):
                pltpu.make_async_copy(g_h.at[0], gbufs.at[j],
                                      sem_ig[j]).wait()
                pltpu.make_async_copy(g_h.at[0], sbufs.at[j],
                                      sem_is[j]).wait()

            def g_issue(j):
                jr = j % RB
                pltpu.async_copy(tbl_h.at[gbufs.at[j]], rows.at[jr],
                                 sem_g[jr])

            def g_wait(j):
                jr = j % RB
                pltpu.make_async_copy(tbl_h.at[gbufs.at[j]], rows.at[jr],
                                      sem_g[jr]).wait()

            def sc_issue(j):
                jr = j % RB
                pltpu.async_copy(rows.at[jr], acc.at[sbufs.at[j]],
                                 sem_sc[jr], add=True)

            def sc_wait(j):
                jr = j % RB
                pltpu.make_async_copy(rows.at[jr], acc.at[sbufs.at[j]],
                                      sem_sc[jr]).wait()

            def steady(i, j):
                idx_wait(j)
                g_issue(j)
                g_wait((j - 1) % 4)
                sc_issue((j - 1) % 4)
                sc_wait((j - 2) % 4)
                idx_issue((j + 2) % 4, i + 2)

            idx_issue(0, 0)
            idx_issue(1, 1)
            idx_wait(0)
            g_issue(0)
            idx_issue(2, 2)
            idx_wait(1)
            g_issue(1)
            idx_issue(3, 3)
            g_wait(0)
            sc_issue(0)
            steady(jnp.int32(2), 2)
            steady(jnp.int32(3), 3)

            def body(k2, _):
                i0 = 4 + 4 * k2
                for j in range(4):
                    steady(i0 + j, j)
                return 0

            lax.fori_loop(0, (n_my - 4) // 4, body, 0)
            nm = n_my
            g_wait((nm - 1) % 4)
            sc_issue((nm - 1) % 4)
            sc_wait((nm - 2) % 4)
            sc_wait((nm - 1) % 4)
            idx_wait(nm % 4)
            idx_wait((nm + 1) % 4)

        zero(S_ACC)
        plsc.subcore_barrier()
        run_pass(tp_h, esg_h, eds_h)
        plsc.subcore_barrier()
        wb = NUM_S // NSUB
        pltpu.sync_copy(acc.at[pl.ds(sid * wb, wb)],
                        out_h.at[cid, pl.ds(sid * wb, wb)])
        plsc.subcore_barrier()
        zero(P_ACC)
        plsc.subcore_barrier()
        run_pass(ts_h, edg_h, ess_h)
        plsc.subcore_barrier()
        wb = NUM_P // NSUB
        pltpu.sync_copy(acc.at[pl.ds(sid * wb, wb)],
                        out_h.at[cid, pl.ds(NUM_S + sid * wb, wb)])

    return k(tbl_p, tbl_s, es_g, ed_g, es_s, ed_s, zeros_acc)


def _label_gather(tp, ts, lsrc2, ldst2):
    gpw = LG // NW

    @functools.partial(
        pl.kernel,
        out_type=[
            jax.ShapeDtypeStruct((NL_PAD, D_H), jnp.bfloat16),
            jax.ShapeDtypeStruct((NL_PAD, D_H), jnp.bfloat16),
        ],
        mesh=_mesh(),
        compiler_params=_sc_params(),
        scratch_types=[
            pltpu.VMEM((2, 128), jnp.int32),
            pltpu.VMEM((2, 128), jnp.int32),
            pltpu.VMEM((2, 128, D_H), jnp.bfloat16),
            pltpu.VMEM((2, 128, D_H), jnp.bfloat16),
        ] + [pltpu.SemaphoreType.DMA] * 8,
    )
    def k(tp_h, ts_h, ls_h, ld_h, po_h, so_h, ib1, ib2, r1, r2, *sems):
        cid = lax.axis_index("c")
        sid = lax.axis_index("s")
        w = cid * NSUB + sid
        si1, si2, sg1, sg2 = (sems[0:2], sems[2:4], sems[4:6], sems[6:8])

        def g(i):
            return w * gpw + jnp.minimum(i, gpw - 1)

        def idx_issue(j, i):
            pltpu.async_copy(ls_h.at[g(i)], ib1.at[j], si1[j])
            pltpu.async_copy(ld_h.at[g(i)], ib2.at[j], si2[j])

        def idx_wait(j):
            pltpu.make_async_copy(ls_h.at[0], ib1.at[j], si1[j]).wait()
            pltpu.make_async_copy(ld_h.at[0], ib2.at[j], si2[j]).wait()

        def g_issue(j):
            pltpu.async_copy(tp_h.at[ib1.at[j]], r1.at[j], sg1[j])
            pltpu.async_copy(ts_h.at[ib2.at[j]], r2.at[j], sg2[j])

        def g_wait(j):
            pltpu.make_async_copy(tp_h.at[ib1.at[j]], r1.at[j], sg1[j]).wait()
            pltpu.make_async_copy(ts_h.at[ib2.at[j]], r2.at[j], sg2[j]).wait()

        def write(j, i):
            pltpu.sync_copy(r1.at[j], po_h.at[pl.ds(g(i) * 128, 128)])
            pltpu.sync_copy(r2.at[j], so_h.at[pl.ds(g(i) * 128, 128)])

        idx_issue(0, 0)
        idx_wait(0)
        g_issue(0)
        idx_issue(1, 1)

        def body(k2, _):
            for j in range(2):
                i = 1 + 2 * k2 + j
                jj = (1 + j) % 2
                idx_wait(jj)
                g_wait((jj + 1) % 2)
                g_issue(jj)
                idx_issue((jj + 1) % 2, i + 1)
                write((jj + 1) % 2, i - 1)
            return 0

        lax.fori_loop(0, (gpw - 1) // 2, body, 0)
        g_wait((gpw - 1) % 2)
        write((gpw - 1) % 2, gpw - 1)
        idx_wait(gpw % 2)

    return k(tp, ts, lsrc2, ldst2)


def _proj(x, W, b, emb, blk):
    n = x.shape[0]

    def body(x_ref, w_ref, b_ref, e_ref, o_ref):
        o_ref[...] = (jnp.dot(x_ref[...], w_ref[...],
                              preferred_element_type=_f32)
                      + b_ref[...] + e_ref[...]).astype(jnp.bfloat16)

    return pl.pallas_call(
        body,
        grid=(n // blk,),
        in_specs=[
            pl.BlockSpec((blk, D_EMB), lambda i: (i, 0)),
            pl.BlockSpec((D_EMB, D_H), lambda i: (0, 0)),
            pl.BlockSpec((1, D_H), lambda i: (0, 0)),
            pl.BlockSpec((blk, D_H), lambda i: (i, 0)),
        ],
        out_specs=pl.BlockSpec((blk, D_H), lambda i: (i, 0)),
        out_shape=jax.ShapeDtypeStruct((n, D_H), jnp.bfloat16),
    )(x, W, b.reshape(1, D_H), emb)


def _sage_post(summ, recip, x_dst, Wl, bl, Wr, relu, blk, off=0):
    nparts = summ.shape[0]
    n = x_dst.shape[0]

    def body(s_ref, r_ref, xd_ref, wl_ref, bl_ref, wr_ref, o_ref):
        s = s_ref[0].astype(_f32)
        for p in range(1, nparts):
            s = s + s_ref[p].astype(_f32)
        agg = s * r_ref[...]
        o = (jnp.dot(agg, wl_ref[...], preferred_element_type=_f32)
             + bl_ref[...]
             + jnp.dot(xd_ref[...].astype(_f32), wr_ref[...],
                       preferred_element_type=_f32))
        o_ref[...] = (jnp.maximum(o, 0.0) if relu else o).astype(jnp.bfloat16)

    return pl.pallas_call(
        body,
        grid=(n // blk,),
        in_specs=[
            pl.BlockSpec((nparts, blk, D_H), lambda i: (0, i + off, 0)),
            pl.BlockSpec((blk, 1), lambda i: (i, 0)),
            pl.BlockSpec((blk, D_H), lambda i: (i, 0)),
            pl.BlockSpec((D_H, D_H), lambda i: (0, 0)),
            pl.BlockSpec((1, D_H), lambda i: (0, 0)),
            pl.BlockSpec((D_H, D_H), lambda i: (0, 0)),
        ],
        out_specs=pl.BlockSpec((blk, D_H), lambda i: (i, 0)),
        out_shape=jax.ShapeDtypeStruct((n, D_H), jnp.bfloat16),
    )(summ, recip, x_dst, Wl, bl.reshape(1, D_H), Wr)


def _classifier(p_rows, s_rows, W1p, W1s, b1, W2, b2, blk):
    n = p_rows.shape[0]

    def body(p_ref, s_ref, w1p_ref, w1s_ref, b1_ref, w2_ref, b2_ref, o_ref):
        h = (jnp.dot(p_ref[...].astype(_f32), w1p_ref[...],
                     preferred_element_type=_f32)
             + jnp.dot(s_ref[...].astype(_f32), w1s_ref[...],
                       preferred_element_type=_f32)
             + b1_ref[...])
        h = jnp.maximum(h, 0.0)
        o_ref[...] = jnp.dot(h, w2_ref[...],
                             preferred_element_type=_f32) + b2_ref[...]

    return pl.pallas_call(
        body,
        grid=(n // blk,),
        in_specs=[
            pl.BlockSpec((blk, D_H), lambda i: (i, 0)),
            pl.BlockSpec((blk, D_H), lambda i: (i, 0)),
            pl.BlockSpec((D_H, D_H), lambda i: (0, 0)),
            pl.BlockSpec((D_H, D_H), lambda i: (0, 0)),
            pl.BlockSpec((1, D_H), lambda i: (0, 0)),
            pl.BlockSpec((D_H, 1), lambda i: (0, 0)),
            pl.BlockSpec((1, 1), lambda i: (0, 0)),
        ],
        out_specs=pl.BlockSpec((blk, 1), lambda i: (i, 0)),
        out_shape=jax.ShapeDtypeStruct((n, 1), _f32),
    )(p_rows, s_rows, W1p, W1s, b1.reshape(1, D_H), W2, b2.reshape(1, 1))


def kernel(x_paper, x_software, node_id_paper, node_id_software, edge_src,
           edge_dst, label_src, label_dst, paper_emb, software_emb, Wp, bp,
           Ws, bs, c1m_Wl, c1m_bl, c1m_Wr, c1r_Wl, c1r_bl, c1r_Wr, c2m_Wl,
           c2m_bl, c2m_Wr, c2r_Wl, c2r_bl, c2r_Wr, cls_W1, cls_b1, cls_W2,
           cls_b2):
    epad = EGP * 128 - NUM_E

    def pad2(a, val):
        return jnp.concatenate(
            [a, jnp.full((epad,), val, jnp.int32)]).reshape(EGP, 128)

    es_g = pad2(edge_src, 0)
    es_s = pad2(edge_src, NUM_P)
    ed_g = pad2(edge_dst, 0)
    ed_s = pad2(edge_dst, NUM_S)

    lpad = NL_PAD - NUM_L
    ls2 = jnp.concatenate(
        [label_src, jnp.zeros((lpad,), jnp.int32)]).reshape(LG, 128)
    ld2 = jnp.concatenate(
        [label_dst, jnp.zeros((lpad,), jnp.int32)]).reshape(LG, 128)

    z_deg = jnp.zeros((DEG_P_PAD,), _f32)
    z_p = jnp.zeros((P_ACC, D_H), jnp.bfloat16)

    hp_part, hs_part = _deg_kernel(es_s, ed_s, z_deg)
    rp_full, rs_full = _recip_kernel(hp_part, hs_part)
    rp = rp_full[:NUM_P].reshape(NUM_P, 1)
    rs = rs_full[:NUM_S].reshape(NUM_S, 1)

    x_p = _proj(x_paper, Wp, bp, paper_emb, 2000)
    x_s = _proj(x_software, Ws, bs, software_emb, 2000)

    s1 = _segsum_layer(x_p, x_s, es_g, ed_g, es_s, ed_s, z_p)
    h_s = _sage_post(s1, rs, x_s, c1m_Wl, c1m_bl, c1m_Wr, True, 2000)
    h_p = _sage_post(s1, rp, x_p, c1r_Wl, c1r_bl, c1r_Wr, True, 2000,
                     off=NUM_S // 2000)

    s2 = _segsum_layer(h_p, h_s, es_g, ed_g, es_s, ed_s, z_p)
    o_s = _sage_post(s2, rs, h_s, c2m_Wl, c2m_bl, c2m_Wr, False, 2000)
    o_p = _sage_post(s2, rp, h_p, c2r_Wl, c2r_bl, c2r_Wr, False, 2000,
                     off=NUM_S // 2000)

    p_rows, s_rows = _label_gather(o_p, o_s, ls2, ld2)
    out = _classifier(p_rows, s_rows, cls_W1[:D_H], cls_W1[D_H:],
                      cls_b1, cls_W2, cls_b2, 1024)
    return out.reshape(-1)[:NUM_L]


# speedup vs baseline: 1.1118x; 1.1118x over previous
import functools

import jax
import jax.numpy as jnp
from jax import lax
from jax.experimental import pallas as pl
from jax.experimental.pallas import tpu as pltpu
from jax.experimental.pallas import tpu_sc as plsc

NUM_P, NUM_S, NUM_E, NUM_L = 50000, 10000, 800000, 200000
D_EMB, D_H = 128, 64

NC, NSUB, L = 2, 16, 16
NW = NC * NSUB

EG = NUM_E // 128
EGP = 6272
DEG_P_PAD = 50176
DEG_S_PAD = 10240
P_ACC = NUM_P + 16
S_ACC = NUM_S + 16
NL_PAD = 200704
LG = NL_PAD // 128

_mesh = lambda: plsc.VectorSubcoreMesh(
    core_axis_name="c", subcore_axis_name="s", num_cores=NC, num_subcores=NSUB)
_f32 = jnp.float32
_sc_params = lambda: pltpu.CompilerParams(
    needs_layout_passes=False, use_tc_tiling_on_sc=False)


def _deg_kernel(src2, dst2, zeros_deg):
    n_my = EGP // NW

    @functools.partial(
        pl.kernel,
        out_type=[
            jax.ShapeDtypeStruct((NW, DEG_P_PAD), _f32),
            jax.ShapeDtypeStruct((NW, DEG_S_PAD), _f32),
        ],
        mesh=_mesh(),
        compiler_params=_sc_params(),
        scratch_types=[
            pltpu.VMEM((DEG_P_PAD,), _f32),
            pltpu.VMEM((DEG_S_PAD,), _f32),
            pltpu.VMEM((2, 128), jnp.int32),
            pltpu.VMEM((2, 128), jnp.int32),
            pltpu.SemaphoreType.DMA,
            pltpu.SemaphoreType.DMA,
            pltpu.SemaphoreType.DMA,
            pltpu.SemaphoreType.DMA,
        ],
    )
    def k(src_h, dst_h, z_h, hp_out, hs_out, hist_p, hist_s, sbuf, dbuf,
          ss0, ss1, ds0, ds1):
        cid = lax.axis_index("c")
        sid = lax.axis_index("s")
        w = cid * NSUB + sid
        ssem = (ss0, ss1)
        dsem = (ds0, ds1)
        pltpu.sync_copy(z_h.at[pl.ds(0, DEG_P_PAD)], hist_p)
        pltpu.sync_copy(z_h.at[pl.ds(0, DEG_S_PAD)], hist_s)
        ones = jnp.ones((L,), _f32)

        def issue(i, j):
            g = jnp.minimum(w + i * NW, EGP - 1)
            pltpu.async_copy(src_h.at[g], sbuf.at[j], ssem[j])
            pltpu.async_copy(dst_h.at[g], dbuf.at[j], dsem[j])

        def wait(j):
            pltpu.make_async_copy(src_h.at[0], sbuf.at[j], ssem[j]).wait()
            pltpu.make_async_copy(dst_h.at[0], dbuf.at[j], dsem[j]).wait()

        issue(0, 0)
        issue(1, 1)

        def body(k2, _):
            for j in range(2):
                i = 2 * k2 + j
                wait(j)
                for v in range(128 // L):
                    plsc.addupdate_scatter(
                        hist_p, [sbuf[j, pl.ds(v * L, L)]], ones)
                    plsc.addupdate_scatter(
                        hist_s, [dbuf[j, pl.ds(v * L, L)]], ones)
                issue(i + 2, j)
            return 0

        lax.fori_loop(0, n_my // 2, body, 0)
        wait(0)
        wait(1)
        pltpu.sync_copy(hist_p, hp_out.at[w])
        pltpu.sync_copy(hist_s, hs_out.at[w])

    return k(src2, dst2, zeros_deg)


def _recip_kernel(parts_p, parts_s):
    cpt_p = DEG_P_PAD // NW
    cpt_s = DEG_S_PAD // NW

    @functools.partial(
        pl.kernel,
        out_type=[
            jax.ShapeDtypeStruct((DEG_P_PAD,), _f32),
            jax.ShapeDtypeStruct((DEG_S_PAD,), _f32),
        ],
        mesh=_mesh(),
        compiler_params=_sc_params(),
        scratch_types=[
            pltpu.VMEM((NW, cpt_p), _f32),
            pltpu.VMEM((cpt_p,), _f32),
            pltpu.VMEM((NW, cpt_s), _f32),
            pltpu.VMEM((cpt_s,), _f32),
        ],
    )
    def k(pp_h, ps_h, op_h, os_h, bufp, rbufp, bufs, rbufs):
        cid = lax.axis_index("c")
        sid = lax.axis_index("s")
        w = cid * NSUB + sid

        for parts_h, out_h, buf, rbuf, cpt in (
                (pp_h, op_h, bufp, rbufp, cpt_p),
                (ps_h, os_h, bufs, rbufs, cpt_s)):
            for j in range(NW):
                pltpu.sync_copy(parts_h.at[j, pl.ds(w * cpt, cpt)], buf.at[j])

            def body(c, _, buf=buf, rbuf=rbuf):
                off = c * L
                s = buf[0, pl.ds(off, L)]
                for j in range(1, NW):
                    s = s + buf[j, pl.ds(off, L)]
                rbuf[pl.ds(off, L)] = 1.0 / jnp.maximum(s, 1.0)
                return 0

            lax.fori_loop(0, cpt // L, body, 0)
            pltpu.sync_copy(rbuf, out_h.at[pl.ds(w * cpt, cpt)])

    return k(parts_p, parts_s)


def _make_segsum(table, gidx2, sidx2, zeros_acc, n_rows):
    n_my = EGP // NW
    acc_alloc = n_rows + 16
    wb_rpt = n_rows // NSUB
    RB = 4
    init_rpt = acc_alloc // NSUB
    dt = jnp.bfloat16

    @functools.partial(
        pl.kernel,
        out_type=jax.ShapeDtypeStruct((NC, n_rows, D_H), dt),
        mesh=_mesh(),
        compiler_params=_sc_params(),
        scratch_types=[
            pltpu.VMEM_SHARED((acc_alloc, D_H), dt),
            pltpu.VMEM((4, 128), jnp.int32),
            pltpu.VMEM((4, 128), jnp.int32),
            pltpu.VMEM((RB, 128, D_H), dt),
        ] + [pltpu.SemaphoreType.DMA] * (8 + 2 * RB),
    )
    def k(tbl_h, g_h, s_h, z_h, out_h, acc, gbufs, sbufs, rows, *sems):
        cid = lax.axis_index("c")
        sid = lax.axis_index("s")
        sem_ig, sem_is = sems[0:4], sems[4:8]
        sem_g, sem_sc = sems[8:8 + RB], sems[8 + RB:8 + 2 * RB]
        w = cid * NSUB + sid

        pltpu.sync_copy(z_h.at[pl.ds(sid * init_rpt, init_rpt)],
                        acc.at[pl.ds(sid * init_rpt, init_rpt)])
        plsc.subcore_barrier()

        def gid(i):
            return jnp.minimum(w + i * NW, EGP - 1)

        def idx_issue(j, i):
            g = gid(i)
            pltpu.async_copy(g_h.at[g], gbufs.at[j], sem_ig[j])
            pltpu.async_copy(s_h.at[g], sbufs.at[j], sem_is[j])

        def idx_wait(j):
            pltpu.make_async_copy(g_h.at[0], gbufs.at[j], sem_ig[j]).wait()
            pltpu.make_async_copy(g_h.at[0], sbufs.at[j], sem_is[j]).wait()

        def g_issue(j):
            jr = j % RB
            pltpu.async_copy(tbl_h.at[gbufs.at[j]], rows.at[jr], sem_g[jr])

        def g_wait(j):
            jr = j % RB
            pltpu.make_async_copy(tbl_h.at[gbufs.at[j]], rows.at[jr],
                                  sem_g[jr]).wait()

        def sc_issue(j):
            jr = j % RB
            pltpu.async_copy(rows.at[jr], acc.at[sbufs.at[j]], sem_sc[jr],
                             add=True)

        def sc_wait(j):
            jr = j % RB
            pltpu.make_async_copy(rows.at[jr], acc.at[sbufs.at[j]],
                                  sem_sc[jr]).wait()

        def steady(i, j):
            idx_wait(j)
            g_issue(j)
            g_wait((j - 1) % 4)
            sc_issue((j - 1) % 4)
            sc_wait((j - 2) % 4)
            idx_issue((j + 2) % 4, i + 2)

        idx_issue(0, 0)
        idx_issue(1, 1)
        idx_wait(0)
        g_issue(0)
        idx_issue(2, 2)
        idx_wait(1)
        g_issue(1)
        idx_issue(3, 3)
        g_wait(0)
        sc_issue(0)
        steady(jnp.int32(2), 2)
        steady(jnp.int32(3), 3)

        def body(k2, _):
            i0 = 4 + 4 * k2
            for j in range(4):
                steady(i0 + j, j)
            return 0

        lax.fori_loop(0, (n_my - 4) // 4, body, 0)
        nm = n_my
        g_wait((nm - 1) % 4)
        sc_issue((nm - 1) % 4)
        sc_wait((nm - 2) % 4)
        sc_wait((nm - 1) % 4)
        idx_wait(nm % 4)
        idx_wait((nm + 1) % 4)
        plsc.subcore_barrier()
        pltpu.sync_copy(acc.at[pl.ds(sid * wb_rpt, wb_rpt)],
                        out_h.at[cid, pl.ds(sid * wb_rpt, wb_rpt)])

    return k(table, gidx2, sidx2, zeros_acc)


def _label_gather(tp, ts, lsrc2, ldst2):
    gpw = LG // NW

    @functools.partial(
        pl.kernel,
        out_type=[
            jax.ShapeDtypeStruct((NL_PAD, D_H), jnp.bfloat16),
            jax.ShapeDtypeStruct((NL_PAD, D_H), jnp.bfloat16),
        ],
        mesh=_mesh(),
        compiler_params=_sc_params(),
        scratch_types=[
            pltpu.VMEM((2, 128), jnp.int32),
            pltpu.VMEM((2, 128), jnp.int32),
            pltpu.VMEM((2, 128, D_H), jnp.bfloat16),
            pltpu.VMEM((2, 128, D_H), jnp.bfloat16),
        ] + [pltpu.SemaphoreType.DMA] * 8,
    )
    def k(tp_h, ts_h, ls_h, ld_h, po_h, so_h, ib1, ib2, r1, r2, *sems):
        cid = lax.axis_index("c")
        sid = lax.axis_index("s")
        w = cid * NSUB + sid
        si1, si2, sg1, sg2 = (sems[0:2], sems[2:4], sems[4:6], sems[6:8])

        def g(i):
            return w * gpw + jnp.minimum(i, gpw - 1)

        def idx_issue(j, i):
            pltpu.async_copy(ls_h.at[g(i)], ib1.at[j], si1[j])
            pltpu.async_copy(ld_h.at[g(i)], ib2.at[j], si2[j])

        def idx_wait(j):
            pltpu.make_async_copy(ls_h.at[0], ib1.at[j], si1[j]).wait()
            pltpu.make_async_copy(ld_h.at[0], ib2.at[j], si2[j]).wait()

        def g_issue(j):
            pltpu.async_copy(tp_h.at[ib1.at[j]], r1.at[j], sg1[j])
            pltpu.async_copy(ts_h.at[ib2.at[j]], r2.at[j], sg2[j])

        def g_wait(j):
            pltpu.make_async_copy(tp_h.at[ib1.at[j]], r1.at[j], sg1[j]).wait()
            pltpu.make_async_copy(ts_h.at[ib2.at[j]], r2.at[j], sg2[j]).wait()

        def write(j, i):
            pltpu.sync_copy(r1.at[j], po_h.at[pl.ds(g(i) * 128, 128)])
            pltpu.sync_copy(r2.at[j], so_h.at[pl.ds(g(i) * 128, 128)])

        idx_issue(0, 0)
        idx_wait(0)
        g_issue(0)
        idx_issue(1, 1)

        def body(k2, _):
            for j in range(2):
                i = 1 + 2 * k2 + j
                jj = (1 + j) % 2
                idx_wait(jj)
                g_wait((jj + 1) % 2)
                g_issue(jj)
                idx_issue((jj + 1) % 2, i + 1)
                write((jj + 1) % 2, i - 1)
            return 0

        lax.fori_loop(0, (gpw - 1) // 2, body, 0)
        g_wait((gpw - 1) % 2)
        write((gpw - 1) % 2, gpw - 1)
        idx_wait(gpw % 2)

    return k(tp, ts, lsrc2, ldst2)


def _proj(x, W, b, emb, blk):
    n = x.shape[0]

    def body(x_ref, w_ref, b_ref, e_ref, o_ref):
        o_ref[...] = (jnp.dot(x_ref[...], w_ref[...],
                              preferred_element_type=_f32)
                      + b_ref[...] + e_ref[...]).astype(jnp.bfloat16)

    return pl.pallas_call(
        body,
        grid=(n // blk,),
        in_specs=[
            pl.BlockSpec((blk, D_EMB), lambda i: (i, 0)),
            pl.BlockSpec((D_EMB, D_H), lambda i: (0, 0)),
            pl.BlockSpec((1, D_H), lambda i: (0, 0)),
            pl.BlockSpec((blk, D_H), lambda i: (i, 0)),
        ],
        out_specs=pl.BlockSpec((blk, D_H), lambda i: (i, 0)),
        out_shape=jax.ShapeDtypeStruct((n, D_H), jnp.bfloat16),
    )(x, W, b.reshape(1, D_H), emb)


def _sage_post(summ, recip, x_dst, Wl, bl, Wr, relu, blk, off=0):
    nparts = summ.shape[0]
    n = x_dst.shape[0]

    def body(s_ref, r_ref, xd_ref, wl_ref, bl_ref, wr_ref, o_ref):
        s = s_ref[0].astype(_f32)
        for p in range(1, nparts):
            s = s + s_ref[p].astype(_f32)
        agg = s * r_ref[...]
        o = (jnp.dot(agg, wl_ref[...], preferred_element_type=_f32)
             + bl_ref[...]
             + jnp.dot(xd_ref[...].astype(_f32), wr_ref[...],
                       preferred_element_type=_f32))
        o_ref[...] = (jnp.maximum(o, 0.0) if relu else o).astype(jnp.bfloat16)

    return pl.pallas_call(
        body,
        grid=(n // blk,),
        in_specs=[
            pl.BlockSpec((nparts, blk, D_H), lambda i: (0, i + off, 0)),
            pl.BlockSpec((blk, 1), lambda i: (i, 0)),
            pl.BlockSpec((blk, D_H), lambda i: (i, 0)),
            pl.BlockSpec((D_H, D_H), lambda i: (0, 0)),
            pl.BlockSpec((1, D_H), lambda i: (0, 0)),
            pl.BlockSpec((D_H, D_H), lambda i: (0, 0)),
        ],
        out_specs=pl.BlockSpec((blk, D_H), lambda i: (i, 0)),
        out_shape=jax.ShapeDtypeStruct((n, D_H), jnp.bfloat16),
    )(summ, recip, x_dst, Wl, bl.reshape(1, D_H), Wr)


def _classifier(p_rows, s_rows, W1p, W1s, b1, W2, b2, blk):
    n = p_rows.shape[0]

    def body(p_ref, s_ref, w1p_ref, w1s_ref, b1_ref, w2_ref, b2_ref, o_ref):
        h = (jnp.dot(p_ref[...].astype(_f32), w1p_ref[...],
                     preferred_element_type=_f32)
             + jnp.dot(s_ref[...].astype(_f32), w1s_ref[...],
                       preferred_element_type=_f32)
             + b1_ref[...])
        h = jnp.maximum(h, 0.0)
        o_ref[...] = jnp.dot(h, w2_ref[...],
                             preferred_element_type=_f32) + b2_ref[...]

    return pl.pallas_call(
        body,
        grid=(n // blk,),
        in_specs=[
            pl.BlockSpec((blk, D_H), lambda i: (i, 0)),
            pl.BlockSpec((blk, D_H), lambda i: (i, 0)),
            pl.BlockSpec((D_H, D_H), lambda i: (0, 0)),
            pl.BlockSpec((D_H, D_H), lambda i: (0, 0)),
            pl.BlockSpec((1, D_H), lambda i: (0, 0)),
            pl.BlockSpec((D_H, 1), lambda i: (0, 0)),
            pl.BlockSpec((1, 1), lambda i: (0, 0)),
        ],
        out_specs=pl.BlockSpec((blk, 1), lambda i: (i, 0)),
        out_shape=jax.ShapeDtypeStruct((n, 1), _f32),
    )(p_rows, s_rows, W1p, W1s, b1.reshape(1, D_H), W2, b2.reshape(1, 1))


def kernel(x_paper, x_software, node_id_paper, node_id_software, edge_src,
           edge_dst, label_src, label_dst, paper_emb, software_emb, Wp, bp,
           Ws, bs, c1m_Wl, c1m_bl, c1m_Wr, c1r_Wl, c1r_bl, c1r_Wr, c2m_Wl,
           c2m_bl, c2m_Wr, c2r_Wl, c2r_bl, c2r_Wr, cls_W1, cls_b1, cls_W2,
           cls_b2):
    epad = EGP * 128 - NUM_E

    def pad2(a, val):
        return jnp.concatenate(
            [a, jnp.full((epad,), val, jnp.int32)]).reshape(EGP, 128)

    es_g = pad2(edge_src, 0)
    es_s = pad2(edge_src, NUM_P)
    ed_g = pad2(edge_dst, 0)
    ed_s = pad2(edge_dst, NUM_S)

    lpad = NL_PAD - NUM_L
    ls2 = jnp.concatenate(
        [label_src, jnp.zeros((lpad,), jnp.int32)]).reshape(LG, 128)
    ld2 = jnp.concatenate(
        [label_dst, jnp.zeros((lpad,), jnp.int32)]).reshape(LG, 128)

    z_deg = jnp.zeros((DEG_P_PAD,), _f32)
    z_p = jnp.zeros((P_ACC, D_H), jnp.bfloat16)
    z_s = jnp.zeros((S_ACC, D_H), jnp.bfloat16)

    hp_part, hs_part = _deg_kernel(es_s, ed_s, z_deg)
    rp_full, rs_full = _recip_kernel(hp_part, hs_part)
    rp = rp_full[:NUM_P].reshape(NUM_P, 1)
    rs = rs_full[:NUM_S].reshape(NUM_S, 1)

    x_p = _proj(x_paper, Wp, bp, paper_emb, 2000)
    x_s = _proj(x_software, Ws, bs, software_emb, 2000)

    ss1 = _make_segsum(x_p, es_g, ed_s, z_s, NUM_S)
    sp1 = _make_segsum(x_s, ed_g, es_s, z_p, NUM_P)
    h_s = _sage_post(ss1, rs, x_s, c1m_Wl, c1m_bl, c1m_Wr, True, 2000)
    h_p = _sage_post(sp1, rp, x_p, c1r_Wl, c1r_bl, c1r_Wr, True, 2000)

    ss2 = _make_segsum(h_p, es_g, ed_s, z_s, NUM_S)
    sp2 = _make_segsum(h_s, ed_g, es_s, z_p, NUM_P)
    o_s = _sage_post(ss2, rs, h_s, c2m_Wl, c2m_bl, c2m_Wr, False, 2000)
    o_p = _sage_post(sp2, rp, h_p, c2r_Wl, c2r_bl, c2r_Wr, False, 2000)

    p_rows, s_rows = _label_gather(o_p, o_s, ls2, ld2)
    out = _classifier(p_rows, s_rows, cls_W1[:D_H], cls_W1[D_H:],
                      cls_b1, cls_W2, cls_b2, 1024)
    return out.reshape(-1)[:NUM_L]
